# Initial kernel scaffold; baseline (speedup 1.0000x reference)
import jax
import jax.numpy as jnp
from jax.experimental import pallas as pl


def kernel(points):
    raise NotImplementedError("write your pallas kernel here")


# speedup vs baseline: 25.9922x; 25.9922x over previous
import jax
import jax.numpy as jnp
from jax.experimental import pallas as pl
from jax.experimental.pallas import tpu as pltpu

_K = 2048


def _fps_kernel(pts_ref, out_ref, dist_ref):
    b, n = dist_ref.shape
    dist_ref[...] = jnp.full((b, n), jnp.inf, dtype=jnp.float32)
    cx0 = pts_ref[0, :, 0:1]
    cy0 = pts_ref[1, :, 0:1]
    cz0 = pts_ref[2, :, 0:1]
    stage0 = jnp.zeros((b, 128), jnp.int32)
    lane = jax.lax.broadcasted_iota(jnp.int32, (b, 128), 1)

    def body(i, carry):
        cx, cy, cz, stage = carry
        px = pts_ref[0]
        py = pts_ref[1]
        pz = pts_ref[2]
        dx = px - cx
        dy = py - cy
        dz = pz - cz
        d = dx * dx + dy * dy + dz * dz
        dist = jnp.minimum(dist_ref[...], d)
        dist_ref[...] = dist
        m = jnp.max(dist, axis=1, keepdims=True)
        iota = jax.lax.broadcasted_iota(jnp.int32, (b, n), 1)
        masked = jnp.where(dist == m, iota, jnp.int32(n))
        idx = jnp.min(masked, axis=1, keepdims=True)
        c = i + 1
        slot = jax.lax.rem(c, 128)
        stage = jnp.where(lane == slot, idx, stage)

        @pl.when(slot == 127)
        def _flush():
            out_ref[jax.lax.div(c, 128)] = stage

        onehot = iota == idx
        zero = jnp.zeros((b, n), jnp.float32)
        ncx = jnp.sum(jnp.where(onehot, px, zero), axis=1, keepdims=True)
        ncy = jnp.sum(jnp.where(onehot, py, zero), axis=1, keepdims=True)
        ncz = jnp.sum(jnp.where(onehot, pz, zero), axis=1, keepdims=True)
        return (ncx, ncy, ncz, stage)

    jax.lax.fori_loop(0, _K - 1, body, (cx0, cy0, cz0, stage0))


def kernel(points):
    b, _, n = points.shape
    pts = jnp.transpose(points, (1, 0, 2))
    out3 = pl.pallas_call(
        _fps_kernel,
        out_shape=jax.ShapeDtypeStruct((_K // 128, b, 128), jnp.int32),
        scratch_shapes=[pltpu.VMEM((b, n), jnp.float32)],
    )(pts)
    return jnp.transpose(out3, (1, 0, 2)).reshape(b, _K)


# speedup vs baseline: 44.5241x; 1.7130x over previous
import jax
import jax.numpy as jnp
from jax.experimental import pallas as pl
from jax.experimental.pallas import tpu as pltpu

_K = 2048
_W = 2048


def _fps_kernel(pts_ref, out_ref, dist_ref):
    b, n = dist_ref.shape
    nchunks = n // _W
    dist_ref[...] = jnp.full((b, n), jnp.inf, dtype=jnp.float32)
    cx0 = pts_ref[0, :, 0:1]
    cy0 = pts_ref[1, :, 0:1]
    cz0 = pts_ref[2, :, 0:1]
    stage0 = jnp.zeros((b, 128), jnp.int32)
    lane = jax.lax.broadcasted_iota(jnp.int32, (b, 128), 1)
    iota_l = jax.lax.broadcasted_iota(jnp.int32, (b, _W), 1)
    zero_w = jnp.zeros((b, _W), jnp.float32)

    def body(i, carry):
        cx, cy, cz, stage = carry
        m = jnp.full((b, 1), -jnp.inf, jnp.float32)
        idx = jnp.zeros((b, 1), jnp.int32)
        wx = jnp.zeros((b, 1), jnp.float32)
        wy = jnp.zeros((b, 1), jnp.float32)
        wz = jnp.zeros((b, 1), jnp.float32)
        for j in range(nchunks):
            sl = slice(j * _W, (j + 1) * _W)
            pxc = pts_ref[0, :, sl]
            pyc = pts_ref[1, :, sl]
            pzc = pts_ref[2, :, sl]
            dx = pxc - cx
            dy = pyc - cy
            dz = pzc - cz
            d = dx * dx + dy * dy + dz * dz
            distc = jnp.minimum(dist_ref[:, sl], d)
            dist_ref[:, sl] = distc
            cm = jnp.max(distc, axis=1, keepdims=True)
            lidx = jnp.min(
                jnp.where(distc == cm, iota_l, jnp.int32(_W)),
                axis=1, keepdims=True)
            oh = iota_l == lidx
            ccx = jnp.sum(jnp.where(oh, pxc, zero_w), axis=1, keepdims=True)
            ccy = jnp.sum(jnp.where(oh, pyc, zero_w), axis=1, keepdims=True)
            ccz = jnp.sum(jnp.where(oh, pzc, zero_w), axis=1, keepdims=True)
            better = cm > m
            m = jnp.where(better, cm, m)
            idx = jnp.where(better, lidx + jnp.int32(j * _W), idx)
            wx = jnp.where(better, ccx, wx)
            wy = jnp.where(better, ccy, wy)
            wz = jnp.where(better, ccz, wz)
        c = i + 1
        slot = jax.lax.rem(c, 128)
        stage = jnp.where(lane == slot, idx, stage)

        @pl.when(slot == 127)
        def _flush():
            out_ref[jax.lax.div(c, 128)] = stage

        return (wx, wy, wz, stage)

    jax.lax.fori_loop(0, _K - 1, body, (cx0, cy0, cz0, stage0))


def kernel(points):
    b, _, n = points.shape
    pts = jnp.transpose(points, (1, 0, 2))
    out3 = pl.pallas_call(
        _fps_kernel,
        out_shape=jax.ShapeDtypeStruct((_K // 128, b, 128), jnp.int32),
        scratch_shapes=[pltpu.VMEM((b, n), jnp.float32)],
    )(pts)
    return jnp.transpose(out3, (1, 0, 2)).reshape(b, _K)


# speedup vs baseline: 53.8435x; 1.2093x over previous
import jax
import jax.numpy as jnp
from jax.experimental import pallas as pl
from jax.experimental.pallas import tpu as pltpu

_K = 2048
_W = 2048


def _fps_kernel(pts_ref, out_ref, dist_ref):
    b, n = dist_ref.shape
    nchunks = n // _W
    dist_ref[...] = jnp.full((b, n), jnp.inf, dtype=jnp.float32)
    cx0 = pts_ref[0, :, 0:1]
    cy0 = pts_ref[1, :, 0:1]
    cz0 = pts_ref[2, :, 0:1]
    stage0 = jnp.zeros((b, 128), jnp.int32)
    lane = jax.lax.broadcasted_iota(jnp.int32, (b, 128), 1)
    iota_f = jax.lax.broadcasted_iota(
        jnp.int32, (b, _W), 1).astype(jnp.float32)
    zero_w = jnp.zeros((b, _W), jnp.float32)

    def body(i, carry):
        cx, cy, cz, stage = carry
        m = jnp.full((b, 1), -jnp.inf, jnp.float32)
        idx = jnp.zeros((b, 1), jnp.int32)
        wx = jnp.zeros((b, 1), jnp.float32)
        wy = jnp.zeros((b, 1), jnp.float32)
        wz = jnp.zeros((b, 1), jnp.float32)
        for j in range(nchunks):
            sl = slice(j * _W, (j + 1) * _W)
            pxc = pts_ref[0, :, sl]
            pyc = pts_ref[1, :, sl]
            pzc = pts_ref[2, :, sl]
            dx = pxc - cx
            dy = pyc - cy
            dz = pzc - cz
            d = dx * dx + dy * dy + dz * dz
            distc = jnp.minimum(dist_ref[:, sl], d)
            dist_ref[:, sl] = distc
            cm = jnp.max(distc, axis=1, keepdims=True)
            lidx = jnp.min(
                jnp.where(distc == cm, iota_f, jnp.float32(_W)),
                axis=1, keepdims=True)
            oh = iota_f == lidx
            ccx = jnp.sum(jnp.where(oh, pxc, zero_w), axis=1, keepdims=True)
            ccy = jnp.sum(jnp.where(oh, pyc, zero_w), axis=1, keepdims=True)
            ccz = jnp.sum(jnp.where(oh, pzc, zero_w), axis=1, keepdims=True)
            better = cm > m
            m = jnp.where(better, cm, m)
            idx = jnp.where(better,
                            lidx.astype(jnp.int32) + jnp.int32(j * _W), idx)
            wx = jnp.where(better, ccx, wx)
            wy = jnp.where(better, ccy, wy)
            wz = jnp.where(better, ccz, wz)
        c = i + 1
        slot = jax.lax.rem(c, 128)
        stage = jnp.where(lane == slot, idx, stage)

        @pl.when(slot == 127)
        def _flush():
            out_ref[jax.lax.div(c, 128)] = stage

        return (wx, wy, wz, stage)

    jax.lax.fori_loop(0, _K - 1, body, (cx0, cy0, cz0, stage0))


def kernel(points):
    b, _, n = points.shape
    pts = jnp.transpose(points, (1, 0, 2))
    out3 = pl.pallas_call(
        _fps_kernel,
        out_shape=jax.ShapeDtypeStruct((_K // 128, b, 128), jnp.int32),
        scratch_shapes=[pltpu.VMEM((b, n), jnp.float32)],
    )(pts)
    return jnp.transpose(out3, (1, 0, 2)).reshape(b, _K)
